# baseline (speedup 1.0000x reference)
import functools
import jax
import jax.numpy as jnp
from jax import lax
from jax.experimental import pallas as pl
from jax.experimental.pallas import tpu as pltpu
from jax.experimental.pallas import tpu_sc as plsc

_SIZE = 128
_DOMAIN = 128
_BATCH = 1024
_ROWS = _BATCH * _SIZE
_NW = 32
_ROWS_W = _ROWS // _NW
_CHUNK = 256
_NCHUNK = _ROWS_W // _CHUNK
_CW = _CHUNK * _DOMAIN

_LN2 = 0.6931471805599453


def _softlog(s):
    bits = lax.bitcast_convert_type(s, jnp.int32)
    e = (bits >> 23) - 127
    mbits = (bits & jnp.int32(0x007FFFFF)) | jnp.int32(0x3F800000)
    m = lax.bitcast_convert_type(mbits, jnp.float32)
    t = (m - 1.0) / (m + 1.0)
    t2 = t * t
    p = 2.0 * t * (1.0 + t2 * (1.0 / 3.0 + t2 * (0.2 + t2 * (1.0 / 7.0))))
    return e.astype(jnp.float32) * _LN2 + p


def _sc_body(idx_hbm, x_hbm, out_hbm, idx_v, xbuf, obuf):
    wid = lax.axis_index("s") * 2 + lax.axis_index("c")
    base = wid * (_ROWS_W * _DOMAIN)
    pltpu.sync_copy(idx_hbm, idx_v)
    lane = lax.iota(jnp.int32, 16)
    neg_inf = jnp.full((16,), -jnp.inf, dtype=jnp.float32)
    zero = jnp.zeros((16,), dtype=jnp.float32)

    def chunk_fn(ci, carry):
        cbase = base + ci * _CW
        pltpu.sync_copy(x_hbm.at[pl.ds(cbase, _CW)], xbuf)

        def strip_fn(si, carry2):
            bvec = lane * _DOMAIN + si * (16 * _DOMAIN)

            def p1(c, ms):
                m0, m1, m2, m3 = ms
                col = 4 * c
                v0 = plsc.load_gather(xbuf, [bvec + col])
                v1 = plsc.load_gather(xbuf, [bvec + (col + 1)])
                v2 = plsc.load_gather(xbuf, [bvec + (col + 2)])
                v3 = plsc.load_gather(xbuf, [bvec + (col + 3)])
                return (jnp.maximum(m0, v0), jnp.maximum(m1, v1),
                        jnp.maximum(m2, v2), jnp.maximum(m3, v3))

            m0, m1, m2, m3 = lax.fori_loop(
                0, _DOMAIN // 4, p1, (neg_inf, neg_inf, neg_inf, neg_inf))
            m = jnp.maximum(jnp.maximum(m0, m1), jnp.maximum(m2, m3))

            def p2(c, ss):
                s0, s1, s2, s3 = ss
                col = 4 * c
                v0 = plsc.load_gather(xbuf, [bvec + col])
                v1 = plsc.load_gather(xbuf, [bvec + (col + 1)])
                v2 = plsc.load_gather(xbuf, [bvec + (col + 2)])
                v3 = plsc.load_gather(xbuf, [bvec + (col + 3)])
                return (s0 + jnp.exp(v0 - m), s1 + jnp.exp(v1 - m),
                        s2 + jnp.exp(v2 - m), s3 + jnp.exp(v3 - m))

            s0, s1, s2, s3 = lax.fori_loop(
                0, _DOMAIN // 4, p2, (zero, zero, zero, zero))
            lse = m + _softlog((s0 + s1) + (s2 + s3))

            def p3(jb, c3):
                vec_i = idx_v[pl.ds(jb * 16, 16)]
                for l in range(16):
                    g = plsc.load_gather(xbuf, [bvec + vec_i[l]])
                    plsc.store_scatter(obuf, [bvec + (jb * 16 + l)], g - lse)
                return c3

            lax.fori_loop(0, _DOMAIN // 16, p3, 0)
            return carry2

        lax.fori_loop(0, _CHUNK // 16, strip_fn, 0)
        pltpu.sync_copy(obuf, out_hbm.at[pl.ds(cbase, _CW)])
        return carry

    lax.fori_loop(0, _NCHUNK, chunk_fn, 0)


def kernel(inputs, log_probs):
    x = log_probs.reshape(_ROWS * _DOMAIN)
    idx = inputs.reshape(_SIZE).astype(jnp.int32)
    mesh = plsc.VectorSubcoreMesh(core_axis_name="c", subcore_axis_name="s")
    run = functools.partial(
        pl.kernel,
        mesh=mesh,
        out_type=jax.ShapeDtypeStruct((_ROWS * _DOMAIN,), jnp.float32),
        scratch_types=[
            pltpu.VMEM((_SIZE,), jnp.int32),
            pltpu.VMEM((_CW,), jnp.float32),
            pltpu.VMEM((_CW,), jnp.float32),
        ],
        compiler_params=pltpu.CompilerParams(needs_layout_passes=False),
    )(_sc_body)
    out = run(idx, x)
    return out.reshape(_BATCH, _SIZE, _DOMAIN)


# speedup vs baseline: 1.0415x; 1.0415x over previous
import functools
import jax
import jax.numpy as jnp
from jax import lax
from jax.experimental import pallas as pl
from jax.experimental.pallas import tpu as pltpu
from jax.experimental.pallas import tpu_sc as plsc

_SIZE = 128
_DOMAIN = 128
_BATCH = 1024
_ROWS = _BATCH * _SIZE
_NW = 32
_ROWS_W = _ROWS // _NW
_CHUNK = 256
_NCHUNK = _ROWS_W // _CHUNK
_CW = _CHUNK * _DOMAIN

_LN2 = 0.6931471805599453


def _softlog(s):
    bits = lax.bitcast_convert_type(s, jnp.int32)
    e = (bits >> 23) - 127
    mbits = (bits & jnp.int32(0x007FFFFF)) | jnp.int32(0x3F800000)
    m = lax.bitcast_convert_type(mbits, jnp.float32)
    t = (m - 1.0) / (m + 1.0)
    t2 = t * t
    p = 2.0 * t * (1.0 + t2 * (1.0 / 3.0 + t2 * (0.2 + t2 * (1.0 / 7.0))))
    return e.astype(jnp.float32) * _LN2 + p


def _sc_body(idx_hbm, x_hbm, out_hbm, idx_v, xbuf, obuf):
    wid = lax.axis_index("s") * 2 + lax.axis_index("c")
    base = wid * (_ROWS_W * _DOMAIN)
    pltpu.sync_copy(idx_hbm, idx_v)
    lane = lax.iota(jnp.int32, 16)
    neg_inf = jnp.full((16,), -jnp.inf, dtype=jnp.float32)
    zero = jnp.zeros((16,), dtype=jnp.float32)

    def chunk_fn(ci, carry):
        cbase = base + ci * _CW
        pltpu.sync_copy(x_hbm.at[pl.ds(cbase, _CW)], xbuf)

        def strip_fn(si, carry2):
            bvec = lane * _DOMAIN + si * (16 * _DOMAIN)

            def p1(c, ms):
                iv, m0, m1, m2, m3 = ms
                v0 = plsc.load_gather(xbuf, [iv])
                v1 = plsc.load_gather(xbuf, [iv + 1])
                v2 = plsc.load_gather(xbuf, [iv + 2])
                v3 = plsc.load_gather(xbuf, [iv + 3])
                return (iv + 4, jnp.maximum(m0, v0), jnp.maximum(m1, v1),
                        jnp.maximum(m2, v2), jnp.maximum(m3, v3))

            _, m0, m1, m2, m3 = lax.fori_loop(
                0, _DOMAIN // 4, p1,
                (bvec, neg_inf, neg_inf, neg_inf, neg_inf))
            m = jnp.maximum(jnp.maximum(m0, m1), jnp.maximum(m2, m3))

            def p2(c, ss):
                iv, s0, s1, s2, s3 = ss
                v0 = plsc.load_gather(xbuf, [iv])
                v1 = plsc.load_gather(xbuf, [iv + 1])
                v2 = plsc.load_gather(xbuf, [iv + 2])
                v3 = plsc.load_gather(xbuf, [iv + 3])
                return (iv + 4, s0 + jnp.exp(v0 - m), s1 + jnp.exp(v1 - m),
                        s2 + jnp.exp(v2 - m), s3 + jnp.exp(v3 - m))

            _, s0, s1, s2, s3 = lax.fori_loop(
                0, _DOMAIN // 4, p2, (bvec, zero, zero, zero, zero))
            lse = m + _softlog((s0 + s1) + (s2 + s3))

            def p3(jb, ov):
                vec_i = idx_v[pl.ds(jb * 16, 16)]
                for l in range(16):
                    g = plsc.load_gather(xbuf, [bvec + vec_i[l]])
                    plsc.store_scatter(obuf, [ov + l], g - lse)
                return ov + 16

            lax.fori_loop(0, _DOMAIN // 16, p3, bvec)
            return carry2

        lax.fori_loop(0, _CHUNK // 16, strip_fn, 0)
        pltpu.sync_copy(obuf, out_hbm.at[pl.ds(cbase, _CW)])
        return carry

    lax.fori_loop(0, _NCHUNK, chunk_fn, 0)


def kernel(inputs, log_probs):
    x = log_probs.reshape(_ROWS * _DOMAIN)
    idx = inputs.reshape(_SIZE).astype(jnp.int32)
    mesh = plsc.VectorSubcoreMesh(core_axis_name="c", subcore_axis_name="s")
    run = functools.partial(
        pl.kernel,
        mesh=mesh,
        out_type=jax.ShapeDtypeStruct((_ROWS * _DOMAIN,), jnp.float32),
        scratch_types=[
            pltpu.VMEM((_SIZE,), jnp.int32),
            pltpu.VMEM((_CW,), jnp.float32),
            pltpu.VMEM((_CW,), jnp.float32),
        ],
        compiler_params=pltpu.CompilerParams(needs_layout_passes=False),
    )(_sc_body)
    out = run(idx, x)
    return out.reshape(_BATCH, _SIZE, _DOMAIN)


# speedup vs baseline: 2.0051x; 1.9253x over previous
import functools
import jax
import jax.numpy as jnp
from jax import lax
from jax.experimental import pallas as pl
from jax.experimental.pallas import tpu as pltpu
from jax.experimental.pallas import tpu_sc as plsc

_SIZE = 128
_DOMAIN = 128
_BATCH = 1024
_ROWS = _BATCH * _SIZE
_NW = 32
_ROWS_W = _ROWS // _NW
_CHUNK = 128
_NCHUNK = _ROWS_W // _CHUNK
_CW = _CHUNK * _DOMAIN
_PAD = _DOMAIN + 1

_LN2 = 0.6931471805599453


def _softlog(s):
    bits = lax.bitcast_convert_type(s, jnp.int32)
    e = (bits >> 23) - 127
    mbits = (bits & jnp.int32(0x007FFFFF)) | jnp.int32(0x3F800000)
    m = lax.bitcast_convert_type(mbits, jnp.float32)
    t = (m - 1.0) / (m + 1.0)
    t2 = t * t
    p = 2.0 * t * (1.0 + t2 * (1.0 / 3.0 + t2 * (0.2 + t2 * (1.0 / 7.0))))
    return e.astype(jnp.float32) * _LN2 + p


def _sc_body(idx_hbm, x_hbm, out_hbm, idx_v, iobuf, xpad, opad):
    wid = lax.axis_index("s") * 2 + lax.axis_index("c")
    base = wid * (_ROWS_W * _DOMAIN)
    pltpu.sync_copy(idx_hbm, idx_v)
    lane = lax.iota(jnp.int32, 16)
    neg_inf = jnp.full((16,), -jnp.inf, dtype=jnp.float32)
    zero = jnp.zeros((16,), dtype=jnp.float32)

    def chunk_fn(ci, carry):
        cbase = base + ci * _CW
        pltpu.sync_copy(x_hbm.at[pl.ds(cbase, _CW)], iobuf)

        def rl_in(r, c0):
            s = r * _DOMAIN
            d = r * _PAD
            for k in range(_DOMAIN // 16):
                xpad[pl.ds(d + 16 * k, 16)] = iobuf[pl.ds(s + 16 * k, 16)]
            return c0

        lax.fori_loop(0, _CHUNK, rl_in, 0)

        def strip_fn(si, c1):
            bvec = lane * _PAD + si * (16 * _PAD)

            def p1(c, ms):
                iv, m0, m1, m2, m3 = ms
                v0 = plsc.load_gather(xpad, [iv])
                v1 = plsc.load_gather(xpad, [iv + 1])
                v2 = plsc.load_gather(xpad, [iv + 2])
                v3 = plsc.load_gather(xpad, [iv + 3])
                return (iv + 4, jnp.maximum(m0, v0), jnp.maximum(m1, v1),
                        jnp.maximum(m2, v2), jnp.maximum(m3, v3))

            _, m0, m1, m2, m3 = lax.fori_loop(
                0, _DOMAIN // 4, p1,
                (bvec, neg_inf, neg_inf, neg_inf, neg_inf))
            m = jnp.maximum(jnp.maximum(m0, m1), jnp.maximum(m2, m3))

            def p2(c, ss):
                iv, s0, s1, s2, s3 = ss
                v0 = plsc.load_gather(xpad, [iv])
                v1 = plsc.load_gather(xpad, [iv + 1])
                v2 = plsc.load_gather(xpad, [iv + 2])
                v3 = plsc.load_gather(xpad, [iv + 3])
                return (iv + 4, s0 + jnp.exp(v0 - m), s1 + jnp.exp(v1 - m),
                        s2 + jnp.exp(v2 - m), s3 + jnp.exp(v3 - m))

            _, s0, s1, s2, s3 = lax.fori_loop(
                0, _DOMAIN // 4, p2, (bvec, zero, zero, zero, zero))
            lse = m + _softlog((s0 + s1) + (s2 + s3))

            def p3(jb, ov):
                vec_i = idx_v[pl.ds(jb * 16, 16)]
                o = ov
                for l in range(16):
                    g = plsc.load_gather(xpad, [bvec + vec_i[l]])
                    plsc.store_scatter(opad, [o], g - lse)
                    o = o + 1
                return o

            lax.fori_loop(0, _DOMAIN // 16, p3, bvec)
            return c1

        lax.fori_loop(0, _CHUNK // 16, strip_fn, 0)

        def rl_out(r, c2):
            s = r * _PAD
            d = r * _DOMAIN
            for k in range(_DOMAIN // 16):
                iobuf[pl.ds(d + 16 * k, 16)] = opad[pl.ds(s + 16 * k, 16)]
            return c2

        lax.fori_loop(0, _CHUNK, rl_out, 0)
        pltpu.sync_copy(iobuf, out_hbm.at[pl.ds(cbase, _CW)])
        return carry

    lax.fori_loop(0, _NCHUNK, chunk_fn, 0)


def kernel(inputs, log_probs):
    x = log_probs.reshape(_ROWS * _DOMAIN)
    idx = inputs.reshape(_SIZE).astype(jnp.int32)
    mesh = plsc.VectorSubcoreMesh(core_axis_name="c", subcore_axis_name="s")
    run = functools.partial(
        pl.kernel,
        mesh=mesh,
        out_type=jax.ShapeDtypeStruct((_ROWS * _DOMAIN,), jnp.float32),
        scratch_types=[
            pltpu.VMEM((_SIZE,), jnp.int32),
            pltpu.VMEM((_CW,), jnp.float32),
            pltpu.VMEM((_CHUNK * _PAD,), jnp.float32),
            pltpu.VMEM((_CHUNK * _PAD,), jnp.float32),
        ],
        compiler_params=pltpu.CompilerParams(needs_layout_passes=False),
    )(_sc_body)
    out = run(idx, x)
    return out.reshape(_BATCH, _SIZE, _DOMAIN)


# speedup vs baseline: 2.4961x; 1.2449x over previous
import functools
import jax
import jax.numpy as jnp
from jax import lax
from jax.experimental import pallas as pl
from jax.experimental.pallas import tpu as pltpu
from jax.experimental.pallas import tpu_sc as plsc

_SIZE = 128
_DOMAIN = 128
_BATCH = 1024
_ROWS = _BATCH * _SIZE
_NW = 32
_ROWS_W = _ROWS // _NW
_CHUNK = 128
_NCHUNK = _ROWS_W // _CHUNK
_CW = _CHUNK * _DOMAIN
_PAD = _DOMAIN + 1

_LN2 = 0.6931471805599453


def _softlog(s):
    bits = lax.bitcast_convert_type(s, jnp.int32)
    e = (bits >> 23) - 127
    mbits = (bits & jnp.int32(0x007FFFFF)) | jnp.int32(0x3F800000)
    m = lax.bitcast_convert_type(mbits, jnp.float32)
    t = (m - 1.0) / (m + 1.0)
    t2 = t * t
    p = 2.0 * t * (1.0 + t2 * (1.0 / 3.0 + t2 * (0.2 + t2 * (1.0 / 7.0))))
    return e.astype(jnp.float32) * _LN2 + p


def _sc_body(idx_hbm, x_hbm, out_hbm, idx_v, iobuf, xpad, opad):
    wid = lax.axis_index("s") * 2 + lax.axis_index("c")
    base = wid * (_ROWS_W * _DOMAIN)
    pltpu.sync_copy(idx_hbm, idx_v)
    lane = lax.iota(jnp.int32, 16)
    neg_inf = jnp.full((16,), -jnp.inf, dtype=jnp.float32)
    zero = jnp.zeros((16,), dtype=jnp.float32)

    def chunk_fn(ci, carry):
        cbase = base + ci * _CW
        pltpu.sync_copy(x_hbm.at[pl.ds(cbase, _CW)], iobuf)

        def rl_in(r, c0):
            for rr in range(2):
                s = (2 * r + rr) * _DOMAIN
                d = (2 * r + rr) * _PAD
                for k in range(_DOMAIN // 16):
                    xpad[pl.ds(d + 16 * k, 16)] = iobuf[pl.ds(s + 16 * k, 16)]
            return c0

        lax.fori_loop(0, _CHUNK // 2, rl_in, 0)

        def strip_fn(si, c1):
            bvec = lane * _PAD + si * (16 * _PAD)

            def p1(c, ms):
                iv, m0, m1, m2, m3 = ms
                v = [plsc.load_gather(xpad, [iv + k]) for k in range(8)]
                return (iv + 8,
                        jnp.maximum(jnp.maximum(m0, v[0]), v[4]),
                        jnp.maximum(jnp.maximum(m1, v[1]), v[5]),
                        jnp.maximum(jnp.maximum(m2, v[2]), v[6]),
                        jnp.maximum(jnp.maximum(m3, v[3]), v[7]))

            _, m0, m1, m2, m3 = lax.fori_loop(
                0, _DOMAIN // 8, p1,
                (bvec, neg_inf, neg_inf, neg_inf, neg_inf))
            m = jnp.maximum(jnp.maximum(m0, m1), jnp.maximum(m2, m3))

            def p2(c, ss):
                iv, s0, s1, s2, s3 = ss
                v = [plsc.load_gather(xpad, [iv + k]) for k in range(8)]
                e = [jnp.exp(vk - m) for vk in v]
                return (iv + 8, s0 + (e[0] + e[4]), s1 + (e[1] + e[5]),
                        s2 + (e[2] + e[6]), s3 + (e[3] + e[7]))

            _, s0, s1, s2, s3 = lax.fori_loop(
                0, _DOMAIN // 8, p2, (bvec, zero, zero, zero, zero))
            lse = m + _softlog((s0 + s1) + (s2 + s3))

            def p3(jb, ov):
                vec_i = idx_v[pl.ds(jb * 16, 16)]
                for l in range(4):
                    g = [plsc.load_gather(xpad, [bvec + vec_i[l + 4 * q]])
                         for q in range(4)]
                    for q in range(4):
                        plsc.store_scatter(opad, [ov + (l + 4 * q)],
                                           g[q] - lse)
                return ov + 16

            lax.fori_loop(0, _DOMAIN // 16, p3, bvec)
            return c1

        lax.fori_loop(0, _CHUNK // 16, strip_fn, 0)

        def rl_out(r, c2):
            for rr in range(2):
                s = (2 * r + rr) * _PAD
                d = (2 * r + rr) * _DOMAIN
                for k in range(_DOMAIN // 16):
                    iobuf[pl.ds(d + 16 * k, 16)] = opad[pl.ds(s + 16 * k, 16)]
            return c2

        lax.fori_loop(0, _CHUNK // 2, rl_out, 0)
        pltpu.sync_copy(iobuf, out_hbm.at[pl.ds(cbase, _CW)])
        return carry

    lax.fori_loop(0, _NCHUNK, chunk_fn, 0)


def kernel(inputs, log_probs):
    x = log_probs.reshape(_ROWS * _DOMAIN)
    idx = inputs.reshape(_SIZE).astype(jnp.int32)
    mesh = plsc.VectorSubcoreMesh(core_axis_name="c", subcore_axis_name="s")
    run = functools.partial(
        pl.kernel,
        mesh=mesh,
        out_type=jax.ShapeDtypeStruct((_ROWS * _DOMAIN,), jnp.float32),
        scratch_types=[
            pltpu.VMEM((_SIZE,), jnp.int32),
            pltpu.VMEM((_CW,), jnp.float32),
            pltpu.VMEM((_CHUNK * _PAD,), jnp.float32),
            pltpu.VMEM((_CHUNK * _PAD,), jnp.float32),
        ],
        compiler_params=pltpu.CompilerParams(needs_layout_passes=False),
    )(_sc_body)
    out = run(idx, x)
    return out.reshape(_BATCH, _SIZE, _DOMAIN)


# speedup vs baseline: 2.8485x; 1.1412x over previous
import functools
import jax
import jax.numpy as jnp
from jax import lax
from jax.experimental import pallas as pl
from jax.experimental.pallas import tpu as pltpu
from jax.experimental.pallas import tpu_sc as plsc

_SIZE = 128
_DOMAIN = 128
_BATCH = 1024
_ROWS = _BATCH * _SIZE
_NW = 32
_ROWS_W = _ROWS // _NW
_CHUNK = 128
_NCHUNK = _ROWS_W // _CHUNK
_CW = _CHUNK * _DOMAIN
_PAD = _DOMAIN + 1

_LN2 = 0.6931471805599453


def _softlog(s):
    bits = lax.bitcast_convert_type(s, jnp.int32)
    e = (bits >> 23) - 127
    mbits = (bits & jnp.int32(0x007FFFFF)) | jnp.int32(0x3F800000)
    m = lax.bitcast_convert_type(mbits, jnp.float32)
    t = (m - 1.0) / (m + 1.0)
    t2 = t * t
    p = 2.0 * t * (1.0 + t2 * (1.0 / 3.0 + t2 * (0.2 + t2 * (1.0 / 7.0))))
    return e.astype(jnp.float32) * _LN2 + p


def _sc_body(idx_hbm, x_hbm, out_hbm, idx_v,
             ibuf0, ibuf1, obuf0, obuf1, xpad, opad,
             isem0, isem1, osem0, osem1):
    wid = lax.axis_index("s") * 2 + lax.axis_index("c")
    base = wid * (_ROWS_W * _DOMAIN)
    pltpu.sync_copy(idx_hbm, idx_v)
    lane = lax.iota(jnp.int32, 16)
    neg_inf = jnp.full((16,), -jnp.inf, dtype=jnp.float32)
    zero = jnp.zeros((16,), dtype=jnp.float32)

    def in_copy(ci, b):
        return pltpu.make_async_copy(
            x_hbm.at[pl.ds(base + ci * _CW, _CW)],
            ibuf0 if b == 0 else ibuf1,
            isem0 if b == 0 else isem1)

    def out_copy(ci, b):
        return pltpu.make_async_copy(
            obuf0 if b == 0 else obuf1,
            out_hbm.at[pl.ds(base + ci * _CW, _CW)],
            osem0 if b == 0 else osem1)

    in_copy(0, 0).start()
    in_copy(1, 1).start()

    def half(p, b):
        ci = 2 * p + b
        iobuf = ibuf0 if b == 0 else ibuf1
        obuf = obuf0 if b == 0 else obuf1
        in_copy(ci, b).wait()

        @pl.when(p > 0)
        def _():
            out_copy(ci - 2, b).wait()

        def rl_in(r, c0):
            for rr in range(2):
                s = (2 * r + rr) * _DOMAIN
                d = (2 * r + rr) * _PAD
                for k in range(_DOMAIN // 16):
                    xpad[pl.ds(d + 16 * k, 16)] = iobuf[pl.ds(s + 16 * k, 16)]
            return c0

        lax.fori_loop(0, _CHUNK // 2, rl_in, 0)

        def strip_fn(si, c1):
            bvec = lane * _PAD + si * (16 * _PAD)

            def p1(c, ms):
                iv, m0, m1, m2, m3 = ms
                v = [plsc.load_gather(xpad, [iv + k]) for k in range(8)]
                return (iv + 8,
                        jnp.maximum(jnp.maximum(m0, v[0]), v[4]),
                        jnp.maximum(jnp.maximum(m1, v[1]), v[5]),
                        jnp.maximum(jnp.maximum(m2, v[2]), v[6]),
                        jnp.maximum(jnp.maximum(m3, v[3]), v[7]))

            _, m0, m1, m2, m3 = lax.fori_loop(
                0, _DOMAIN // 8, p1,
                (bvec, neg_inf, neg_inf, neg_inf, neg_inf))
            m = jnp.maximum(jnp.maximum(m0, m1), jnp.maximum(m2, m3))

            def p2(c, ss):
                iv, s0, s1, s2, s3 = ss
                v = [plsc.load_gather(xpad, [iv + k]) for k in range(8)]
                e = [jnp.exp(vk - m) for vk in v]
                return (iv + 8, s0 + (e[0] + e[4]), s1 + (e[1] + e[5]),
                        s2 + (e[2] + e[6]), s3 + (e[3] + e[7]))

            _, s0, s1, s2, s3 = lax.fori_loop(
                0, _DOMAIN // 8, p2, (bvec, zero, zero, zero, zero))
            lse = m + _softlog((s0 + s1) + (s2 + s3))

            def p3(jb, ov):
                vec_i = idx_v[pl.ds(jb * 16, 16)]
                for l in range(4):
                    g = [plsc.load_gather(xpad, [bvec + vec_i[l + 4 * q]])
                         for q in range(4)]
                    for q in range(4):
                        plsc.store_scatter(opad, [ov + (l + 4 * q)],
                                           g[q] - lse)
                return ov + 16

            lax.fori_loop(0, _DOMAIN // 16, p3, bvec)
            return c1

        lax.fori_loop(0, _CHUNK // 16, strip_fn, 0)

        def rl_out(r, c2):
            for rr in range(2):
                s = (2 * r + rr) * _PAD
                d = (2 * r + rr) * _DOMAIN
                for k in range(_DOMAIN // 16):
                    obuf[pl.ds(d + 16 * k, 16)] = opad[pl.ds(s + 16 * k, 16)]
            return c2

        lax.fori_loop(0, _CHUNK // 2, rl_out, 0)
        out_copy(ci, b).start()

        @pl.when(p < _NCHUNK // 2 - 1)
        def _():
            in_copy(ci + 2, b).start()

    def pair_fn(p, carry):
        half(p, 0)
        half(p, 1)
        return carry

    lax.fori_loop(0, _NCHUNK // 2, pair_fn, 0)
    out_copy(_NCHUNK - 2, 0).wait()
    out_copy(_NCHUNK - 1, 1).wait()


def kernel(inputs, log_probs):
    x = log_probs.reshape(_ROWS * _DOMAIN)
    idx = inputs.reshape(_SIZE).astype(jnp.int32)
    mesh = plsc.VectorSubcoreMesh(core_axis_name="c", subcore_axis_name="s")
    run = functools.partial(
        pl.kernel,
        mesh=mesh,
        out_type=jax.ShapeDtypeStruct((_ROWS * _DOMAIN,), jnp.float32),
        scratch_types=[
            pltpu.VMEM((_SIZE,), jnp.int32),
            pltpu.VMEM((_CW,), jnp.float32),
            pltpu.VMEM((_CW,), jnp.float32),
            pltpu.VMEM((_CW,), jnp.float32),
            pltpu.VMEM((_CW,), jnp.float32),
            pltpu.VMEM((_CHUNK * _PAD,), jnp.float32),
            pltpu.VMEM((_CHUNK * _PAD,), jnp.float32),
            pltpu.SemaphoreType.DMA,
            pltpu.SemaphoreType.DMA,
            pltpu.SemaphoreType.DMA,
            pltpu.SemaphoreType.DMA,
        ],
        compiler_params=pltpu.CompilerParams(needs_layout_passes=False),
    )(_sc_body)
    out = run(idx, x)
    return out.reshape(_BATCH, _SIZE, _DOMAIN)


# speedup vs baseline: 2.9846x; 1.0478x over previous
import functools
import jax
import jax.numpy as jnp
from jax import lax
from jax.experimental import pallas as pl
from jax.experimental.pallas import tpu as pltpu
from jax.experimental.pallas import tpu_sc as plsc

_SIZE = 128
_DOMAIN = 128
_BATCH = 1024
_ROWS = _BATCH * _SIZE
_NW = 32
_ROWS_W = _ROWS // _NW
_CHUNK = 128
_NCHUNK = _ROWS_W // _CHUNK
_CW = _CHUNK * _DOMAIN
_PAD = _DOMAIN + 1

_LN2 = 0.6931471805599453


def _softlog(s):
    bits = lax.bitcast_convert_type(s, jnp.int32)
    e = (bits >> 23) - 127
    mbits = (bits & jnp.int32(0x007FFFFF)) | jnp.int32(0x3F800000)
    m = lax.bitcast_convert_type(mbits, jnp.float32)
    t = (m - 1.0) / (m + 1.0)
    t2 = t * t
    p = 2.0 * t * (1.0 + t2 * (1.0 / 3.0 + t2 * (0.2 + t2 * (1.0 / 7.0))))
    return e.astype(jnp.float32) * _LN2 + p


def _sc_body(idx_hbm, x_hbm, out_hbm, idx_v,
             ibuf0, ibuf1, obuf0, obuf1, xpad, opad,
             isem0, isem1, osem0, osem1):
    wid = lax.axis_index("s") * 2 + lax.axis_index("c")
    base = wid * (_ROWS_W * _DOMAIN)
    pltpu.sync_copy(idx_hbm, idx_v)
    lane = lax.iota(jnp.int32, 16)
    neg_inf = jnp.full((16,), -jnp.inf, dtype=jnp.float32)
    zero = jnp.zeros((16,), dtype=jnp.float32)

    def in_copy(ci, b):
        return pltpu.make_async_copy(
            x_hbm.at[pl.ds(base + ci * _CW, _CW)],
            ibuf0 if b == 0 else ibuf1,
            isem0 if b == 0 else isem1)

    def out_copy(ci, b):
        return pltpu.make_async_copy(
            obuf0 if b == 0 else obuf1,
            out_hbm.at[pl.ds(base + ci * _CW, _CW)],
            osem0 if b == 0 else osem1)

    in_copy(0, 0).start()
    in_copy(1, 1).start()

    def half(p, b):
        ci = 2 * p + b
        iobuf = ibuf0 if b == 0 else ibuf1
        obuf = obuf0 if b == 0 else obuf1
        in_copy(ci, b).wait()

        @pl.when(p > 0)
        def _():
            out_copy(ci - 2, b).wait()

        def rl_in(r, c0):
            for rr in range(4):
                s = (4 * r + rr) * _DOMAIN
                d = (4 * r + rr) * _PAD
                for k in range(_DOMAIN // 16):
                    xpad[pl.ds(d + 16 * k, 16)] = iobuf[pl.ds(s + 16 * k, 16)]
            return c0

        lax.fori_loop(0, _CHUNK // 4, rl_in, 0)

        def strip_fn(si, c1):
            bvec = lane * _PAD + si * (16 * _PAD)

            def p1(c, ms):
                iv, m0, m1, m2, m3 = ms
                v = [plsc.load_gather(xpad, [iv + k]) for k in range(8)]
                return (iv + 8,
                        jnp.maximum(jnp.maximum(m0, v[0]), v[4]),
                        jnp.maximum(jnp.maximum(m1, v[1]), v[5]),
                        jnp.maximum(jnp.maximum(m2, v[2]), v[6]),
                        jnp.maximum(jnp.maximum(m3, v[3]), v[7]))

            _, m0, m1, m2, m3 = lax.fori_loop(
                0, _DOMAIN // 8, p1,
                (bvec, neg_inf, neg_inf, neg_inf, neg_inf))
            m = jnp.maximum(jnp.maximum(m0, m1), jnp.maximum(m2, m3))

            def p2(c, ss):
                iv = ss[0]
                v = [plsc.load_gather(xpad, [iv + k]) for k in range(8)]
                e = [jnp.exp(vk - m) for vk in v]
                return (iv + 8,) + tuple(ss[1 + k] + e[k] for k in range(8))

            s8 = lax.fori_loop(
                0, _DOMAIN // 8, p2, (bvec,) + (zero,) * 8)[1:]
            s = ((s8[0] + s8[1]) + (s8[2] + s8[3])) + \
                ((s8[4] + s8[5]) + (s8[6] + s8[7]))
            lse = m + _softlog(s)

            def p3(jb, ov):
                vec_i = idx_v[pl.ds(jb * 16, 16)]
                g = [plsc.load_gather(xpad, [bvec + vec_i[q]])
                     for q in range(8)]
                for q in range(8):
                    plsc.store_scatter(opad, [ov + q], g[q] - lse)
                g = [plsc.load_gather(xpad, [bvec + vec_i[8 + q]])
                     for q in range(8)]
                for q in range(8):
                    plsc.store_scatter(opad, [ov + (8 + q)], g[q] - lse)
                return ov + 16

            lax.fori_loop(0, _DOMAIN // 16, p3, bvec)
            return c1

        lax.fori_loop(0, _CHUNK // 16, strip_fn, 0)

        def rl_out(r, c2):
            for rr in range(4):
                s = (4 * r + rr) * _PAD
                d = (4 * r + rr) * _DOMAIN
                for k in range(_DOMAIN // 16):
                    obuf[pl.ds(d + 16 * k, 16)] = opad[pl.ds(s + 16 * k, 16)]
            return c2

        lax.fori_loop(0, _CHUNK // 4, rl_out, 0)
        out_copy(ci, b).start()

        @pl.when(p < _NCHUNK // 2 - 1)
        def _():
            in_copy(ci + 2, b).start()

    def pair_fn(p, carry):
        half(p, 0)
        half(p, 1)
        return carry

    lax.fori_loop(0, _NCHUNK // 2, pair_fn, 0)
    out_copy(_NCHUNK - 2, 0).wait()
    out_copy(_NCHUNK - 1, 1).wait()


def kernel(inputs, log_probs):
    x = log_probs.reshape(_ROWS * _DOMAIN)
    idx = inputs.reshape(_SIZE).astype(jnp.int32)
    mesh = plsc.VectorSubcoreMesh(core_axis_name="c", subcore_axis_name="s")
    run = functools.partial(
        pl.kernel,
        mesh=mesh,
        out_type=jax.ShapeDtypeStruct((_ROWS * _DOMAIN,), jnp.float32),
        scratch_types=[
            pltpu.VMEM((_SIZE,), jnp.int32),
            pltpu.VMEM((_CW,), jnp.float32),
            pltpu.VMEM((_CW,), jnp.float32),
            pltpu.VMEM((_CW,), jnp.float32),
            pltpu.VMEM((_CW,), jnp.float32),
            pltpu.VMEM((_CHUNK * _PAD,), jnp.float32),
            pltpu.VMEM((_CHUNK * _PAD,), jnp.float32),
            pltpu.SemaphoreType.DMA,
            pltpu.SemaphoreType.DMA,
            pltpu.SemaphoreType.DMA,
            pltpu.SemaphoreType.DMA,
        ],
        compiler_params=pltpu.CompilerParams(needs_layout_passes=False),
    )(_sc_body)
    out = run(idx, x)
    return out.reshape(_BATCH, _SIZE, _DOMAIN)
